# baseline (speedup 1.0000x reference)
import jax
import jax.numpy as jnp
from jax.experimental import pallas as pl
from jax.experimental.pallas import tpu as pltpu

R = 20000
D = 1024
K = 117
BR = 1000
NBUF = 6
NSTEP = R // BR


def _mm_kernel(x_hbm, wt_ref, b_ref, o_ref, xbuf, sems):
    i = pl.program_id(0)

    def issue(step, buf):
        pltpu.make_async_copy(
            x_hbm.at[pl.ds(step * BR, BR), :], xbuf.at[buf], sems.at[buf]
        ).start()

    @pl.when(i == 0)
    def _prologue():
        for j in range(NBUF):
            issue(j, j)

    buf = jax.lax.rem(i, NBUF)
    pltpu.make_async_copy(
        x_hbm.at[pl.ds(i * BR, BR), :], xbuf.at[buf], sems.at[buf]
    ).wait()
    acc = jax.lax.dot_general(
        xbuf[buf], wt_ref[...],
        dimension_numbers=(((1,), (0,)), ((), ())),
        preferred_element_type=jnp.float32,
    )
    o_ref[...] = acc + b_ref[...]

    @pl.when(i + NBUF < NSTEP)
    def _refill():
        issue(i + NBUF, buf)


def kernel(x, W, b):
    wt = W.T
    bp = b.reshape(1, K)
    return pl.pallas_call(
        _mm_kernel,
        grid=(NSTEP,),
        in_specs=[
            pl.BlockSpec(memory_space=pl.ANY),
            pl.BlockSpec((D, K), lambda i: (0, 0)),
            pl.BlockSpec((1, K), lambda i: (0, 0)),
        ],
        out_specs=pl.BlockSpec((BR, K), lambda i: (i, 0)),
        out_shape=jax.ShapeDtypeStruct((R, K), jnp.float32),
        scratch_shapes=[
            pltpu.VMEM((NBUF, BR, D), jnp.float32),
            pltpu.SemaphoreType.DMA((NBUF,)),
        ],
        compiler_params=pltpu.CompilerParams(
            dimension_semantics=("arbitrary",),
        ),
    )(x, wt, bp)
